# baseline (speedup 1.0000x reference)
import functools

import jax
import jax.numpy as jnp
from jax import lax
from jax.experimental import pallas as pl
from jax.experimental.pallas import tpu as pltpu
from jax.experimental.pallas import tpu_sc as plsc

NUM_EMB = 1000000
D = 64
W = D // 2
B = 16384
NC = 2
NS = 16
NW = NC * NS
BPW = B // NW
NCHUNK = 4
CR = BPW // NCHUNK


def _sc_body(idx_hbm, table_hbm, out_hbm, idx_v, rows_v, out_v,
             g0, g1, g2, g3, w0, w1, w2, w3):
    gsems = (g0, g1, g2, g3)
    wsems = (w0, w1, w2, w3)
    wid = lax.axis_index("s") * NC + lax.axis_index("c")
    base = wid * BPW

    for c in range(NCHUNK):
        pltpu.sync_copy(idx_hbm.at[pl.ds(base + c * CR, CR)], idx_v.at[c])

    gd = [
        pltpu.async_copy(table_hbm.at[idx_v.at[c]],
                         rows_v.at[pl.ds(c * CR, CR)], gsems[c])
        for c in range(NCHUNK)
    ]

    iot = lax.iota(jnp.int32, 16)
    ecol0 = iot * 2
    half_c = jnp.full((16,), 0x7FFF, jnp.uint32)
    one_c = jnp.full((16,), 1, jnp.uint32)
    himask = jnp.full((16,), 0xFFFF0000, jnp.uint32)
    s16 = jnp.full((16,), 16, jnp.uint32)

    wd = []
    for c in range(NCHUNK):
        gd[c].wait()

        def row_body(r, carry, c=c):
            row = c * CR + r
            rs = jnp.full((16,), row, jnp.int32)
            for half in range(2):
                ce = ecol0 + (half * 32)
                co = ce + 1
                ev = plsc.load_gather(rows_v, [rs, ce])
                od = plsc.load_gather(rows_v, [rs, co])
                ue = plsc.bitcast(ev, jnp.uint32)
                uo = plsc.bitcast(od, jnp.uint32)
                te = ue + half_c + ((ue >> s16) & one_c)
                to = uo + half_c + ((uo >> s16) & one_c)
                word = (te >> s16) | (to & himask)
                out_v[row, pl.ds(half * 16, 16)] = word
            return carry

        lax.fori_loop(0, CR, row_body, 0)
        wd.append(pltpu.async_copy(out_v.at[pl.ds(c * CR, CR)],
                                   out_hbm.at[pl.ds(base + c * CR, CR)],
                                   wsems[c]))
    for d in wd:
        d.wait()


_sc_gather_cast = functools.partial(
    pl.kernel,
    mesh=plsc.VectorSubcoreMesh(core_axis_name="c", subcore_axis_name="s"),
    out_type=jax.ShapeDtypeStruct((B, W), jnp.uint32),
    scratch_types=[
        pltpu.VMEM((NCHUNK, CR), jnp.int32),
        pltpu.VMEM((BPW, D), jnp.float32),
        pltpu.VMEM((BPW, W), jnp.uint32),
    ] + [pltpu.SemaphoreType.DMA] * 8,
    compiler_params=pltpu.CompilerParams(
        needs_layout_passes=False, use_tc_tiling_on_sc=False),
)(_sc_body)


def kernel(inputs, weights):
    packed = _sc_gather_cast(inputs, weights)
    return lax.bitcast_convert_type(packed, jnp.bfloat16).reshape(B, D)


# speedup vs baseline: 1.6481x; 1.6481x over previous
import functools

import jax
import jax.numpy as jnp
from jax import lax
from jax.experimental import pallas as pl
from jax.experimental.pallas import tpu as pltpu
from jax.experimental.pallas import tpu_sc as plsc

NUM_EMB = 1000000
D = 64
W = D // 2
B = 16384
NC = 2
NS = 16
NW = NC * NS
BPW = B // NW
NCHUNK = 4
CR = BPW // NCHUNK


def _sc_body(idx_hbm, table_hbm, out_hbm, idx_v, rows_v, out_v,
             gs0, gs1, wsem):
    wid = lax.axis_index("s") * NC + lax.axis_index("c")
    base = wid * BPW

    pltpu.sync_copy(idx_hbm.at[pl.ds(base, BPW)], idx_v)

    gsems = (gs0, gs1)
    iot = lax.iota(jnp.int32, 16)
    zeros = iot * 0

    def fire(c):
        sem = gsems[c % 2]

        def slab_dma(t, carry):
            row0 = c * CR + t * 16
            vec = idx_v[pl.ds(row0, 16)]
            for k in range(16):
                s = jnp.max(jnp.where(iot == k, vec, zeros))
                pltpu.async_copy(table_hbm.at[pl.ds(s, 1)],
                                 rows_v.at[pl.ds(row0 + k, 1)], sem)
            return carry

        lax.fori_loop(0, CR // 16, slab_dma, 0)

    def drain(c):
        pltpu.make_async_copy(table_hbm.at[pl.ds(0, CR)],
                              rows_v.at[pl.ds(c * CR, CR)],
                              gsems[c % 2]).wait()

    ecol0 = iot * 2
    half_c = jnp.full((16,), 0x7FFF, jnp.uint32)
    one_c = jnp.full((16,), 1, jnp.uint32)
    himask = jnp.full((16,), 0xFFFF0000, jnp.uint32)
    s16 = jnp.full((16,), 16, jnp.uint32)

    fire(0)
    wd = []
    for c in range(NCHUNK):
        if c + 1 < NCHUNK:
            fire(c + 1)
        drain(c)

        def row_body(r, carry, c=c):
            row = c * CR + r
            rs = jnp.full((16,), row, jnp.int32)
            for half in range(2):
                ce = ecol0 + (half * 32)
                co = ce + 1
                ev = plsc.load_gather(rows_v, [rs, ce])
                od = plsc.load_gather(rows_v, [rs, co])
                ue = plsc.bitcast(ev, jnp.uint32)
                uo = plsc.bitcast(od, jnp.uint32)
                te = ue + half_c + ((ue >> s16) & one_c)
                to = uo + half_c + ((uo >> s16) & one_c)
                word = (te >> s16) | (to & himask)
                out_v[pl.ds(row * W + half * 16, 16)] = (
                    plsc.bitcast(word, jnp.int32))
            return carry

        lax.fori_loop(0, CR, row_body, 0, unroll=2)
        wd.append(pltpu.async_copy(out_v.at[pl.ds(c * CR * W, CR * W)],
                                   out_hbm.at[pl.ds((base + c * CR) * W,
                                                    CR * W)],
                                   wsem))
    for d in wd:
        d.wait()


_sc_gather_cast = functools.partial(
    pl.kernel,
    mesh=plsc.VectorSubcoreMesh(core_axis_name="c", subcore_axis_name="s"),
    out_type=jax.ShapeDtypeStruct((B * W,), jnp.int32),
    scratch_types=[
        pltpu.VMEM((BPW,), jnp.int32),
        pltpu.VMEM((BPW, D), jnp.float32),
        pltpu.VMEM((BPW * W,), jnp.int32),
        pltpu.SemaphoreType.DMA,
        pltpu.SemaphoreType.DMA,
        pltpu.SemaphoreType.DMA,
    ],
    compiler_params=pltpu.CompilerParams(
        needs_layout_passes=False, use_tc_tiling_on_sc=True),
)(_sc_body)


def kernel(inputs, weights):
    packed = _sc_gather_cast(inputs, weights)
    return lax.bitcast_convert_type(packed, jnp.bfloat16).reshape(B, D)
